# baseline (speedup 1.0000x reference)
import functools

import jax
import jax.numpy as jnp
from jax.experimental import pallas as pl
from jax.experimental.pallas import tpu as pltpu

_NMS_PRE = 1024
_NMS_POST = 256
_NMS_THRESH = 0.1
_B = 4
_BLK = 128
_NBLK = _NMS_PRE // _BLK


def _nms_body(a_ref, at_ref, o_ref, sup_ref):
    a = a_ref[0]
    at = at_ref[0]

    xc = at[0:1, :]
    yc = at[1:2, :]
    dxc = jnp.abs(at[3:4, :])
    dyc = jnp.abs(at[4:5, :])
    x1c = xc - dxc * 0.5
    x2c = xc + dxc * 0.5
    y1c = yc - dyc * 0.5
    y2c = yc + dyc * 0.5
    areac = dxc * dyc
    scorec = at[7:8, :]

    col_id = jax.lax.broadcasted_iota(jnp.int32, (_BLK, _NMS_PRE), 1)

    for r in range(_NBLK):
        base = r * _BLK
        xr = a[base:base + _BLK, 0:1]
        yr = a[base:base + _BLK, 1:2]
        dxr = jnp.abs(a[base:base + _BLK, 3:4])
        dyr = jnp.abs(a[base:base + _BLK, 4:5])
        x1r = xr - dxr * 0.5
        x2r = xr + dxr * 0.5
        y1r = yr - dyr * 0.5
        y2r = yr + dyr * 0.5
        arear = dxr * dyr

        iw = jnp.clip(jnp.minimum(x2r, x2c) - jnp.maximum(x1r, x1c), 0.0)
        ih = jnp.clip(jnp.minimum(y2r, y2c) - jnp.maximum(y1r, y1c), 0.0)
        inter = iw * ih
        union = arear + areac - inter
        iou = inter / jnp.clip(union, 1e-6)
        row_id = jax.lax.broadcasted_iota(jnp.int32, (_BLK, _NMS_PRE), 0) + base
        sup = jnp.where((iou > _NMS_THRESH) & (col_id > row_id), 1.0, 0.0)
        sup_ref[base:base + _BLK, :] = sup

    it8 = (jax.lax.broadcasted_iota(jnp.int32, (8, _NMS_PRE), 1)
           + jax.lax.broadcasted_iota(jnp.int32, (8, _NMS_PRE), 0))
    ones = jnp.where(it8 >= 0, 1.0, 0.0)

    def w_cond(c):
        k_prev, k, it = c
        return (it < _NMS_PRE + 2) & (jnp.sum(jnp.abs(k - k_prev)) > 0.0)

    def w_body(c):
        _, k, it = c
        supp = jnp.dot(k, sup_ref[:, :], preferred_element_type=jnp.float32)
        k_new = jnp.where(supp < 0.5, 1.0, 0.0)
        return k, k_new, it + 1

    _, keep8, _ = jax.lax.while_loop(
        w_cond, w_body, (-ones, ones, jnp.int32(0)))
    keep = keep8[0:1, :]

    valid = jnp.where(scorec != -jnp.inf, 1.0, 0.0)
    keep_f = keep * valid

    ri = jax.lax.broadcasted_iota(jnp.int32, (_NMS_PRE, _NMS_PRE), 0)
    ci = jax.lax.broadcasted_iota(jnp.int32, (_NMS_PRE, _NMS_PRE), 1)
    tri = jnp.where(ri <= ci, 1.0, 0.0)
    pos = jnp.dot(keep_f, tri, preferred_element_type=jnp.float32)

    out_r = jax.lax.broadcasted_iota(jnp.int32, (_NMS_POST, _NMS_PRE), 0)
    posi = (pos - 1.0).astype(jnp.int32)
    sel = jnp.where(posi == out_r, 1.0, 0.0) * keep_f

    score_clean = jnp.where(a[:, 7:8] != -jnp.inf, a[:, 7:8], 0.0)
    a_mm = jnp.concatenate([a[:, 0:7], score_clean], axis=1)
    o_ref[0] = jnp.dot(sel, a_mm, preferred_element_type=jnp.float32)


@functools.partial(jax.jit, static_argnames=("interpret",))
def _nms_pallas(a, at, interpret=False):
    return pl.pallas_call(
        _nms_body,
        grid=(_B,),
        in_specs=[
            pl.BlockSpec((1, _NMS_PRE, 8), lambda b: (b, 0, 0)),
            pl.BlockSpec((1, 8, _NMS_PRE), lambda b: (b, 0, 0)),
        ],
        out_specs=pl.BlockSpec((1, _NMS_POST, 8), lambda b: (b, 0, 0)),
        out_shape=jax.ShapeDtypeStruct((_B, _NMS_POST, 8), jnp.float32),
        scratch_shapes=[
            pltpu.VMEM((_NMS_PRE, _NMS_PRE), jnp.float32),
        ],
        interpret=interpret,
    )(a, at)


def kernel(batch_box_preds, batch_cls_scores, batch_index, batch_size, interpret=False):
    bids = jnp.arange(_B, dtype=batch_index.dtype)
    masks = (batch_index[None, :] == bids[:, None]) & (bids[:, None] < batch_size)
    masked = jnp.where(masks, batch_cls_scores[None, :], -jnp.inf)
    top_scores, top_idx = jax.lax.top_k(masked, _NMS_PRE)
    boxes_sel = jnp.take(batch_box_preds, top_idx.reshape(-1), axis=0)
    boxes_sel = boxes_sel.reshape(_B, _NMS_PRE, 7)
    a = jnp.concatenate([boxes_sel, top_scores[..., None]], axis=-1)
    at = jnp.swapaxes(a, 1, 2)
    return _nms_pallas(a, at, interpret=interpret)


# speedup vs baseline: 3.1339x; 3.1339x over previous
import functools

import jax
import jax.numpy as jnp
from jax.experimental import pallas as pl
from jax.experimental.pallas import tpu as pltpu

_NMS_PRE = 1024
_NMS_POST = 256
_NMS_THRESH = 0.1
_B = 4
_BLK = 128
_NBLK = _NMS_PRE // _BLK


def _nms_body(a_ref, at_ref, o_ref, sup_ref):
    a = a_ref[0]
    at = at_ref[0]

    xc = at[0:1, :]
    yc = at[1:2, :]
    dxc = jnp.abs(at[3:4, :])
    dyc = jnp.abs(at[4:5, :])
    x1c = xc - dxc * 0.5
    x2c = xc + dxc * 0.5
    y1c = yc - dyc * 0.5
    y2c = yc + dyc * 0.5
    areac = dxc * dyc
    scorec = at[7:8, :]

    col_id = jax.lax.broadcasted_iota(jnp.int32, (_BLK, _NMS_PRE), 1)

    for r in range(_NBLK):
        base = r * _BLK
        xr = a[base:base + _BLK, 0:1]
        yr = a[base:base + _BLK, 1:2]
        dxr = jnp.abs(a[base:base + _BLK, 3:4])
        dyr = jnp.abs(a[base:base + _BLK, 4:5])
        x1r = xr - dxr * 0.5
        x2r = xr + dxr * 0.5
        y1r = yr - dyr * 0.5
        y2r = yr + dyr * 0.5
        arear = dxr * dyr

        iw = jnp.clip(jnp.minimum(x2r, x2c) - jnp.maximum(x1r, x1c), 0.0)
        ih = jnp.clip(jnp.minimum(y2r, y2c) - jnp.maximum(y1r, y1c), 0.0)
        inter = iw * ih
        union = arear + areac - inter
        iou = inter / jnp.clip(union, 1e-6)
        row_id = jax.lax.broadcasted_iota(jnp.int32, (_BLK, _NMS_PRE), 0) + base
        sup = jnp.where((iou > _NMS_THRESH) & (col_id > row_id), 1.0, 0.0)
        sup_ref[base:base + _BLK, :] = sup

    it8 = (jax.lax.broadcasted_iota(jnp.int32, (8, _NMS_PRE), 1)
           + jax.lax.broadcasted_iota(jnp.int32, (8, _NMS_PRE), 0))
    ones = jnp.where(it8 >= 0, 1.0, 0.0)

    def w_cond(c):
        k_prev, k, it = c
        return (it < _NMS_PRE + 2) & (jnp.sum(jnp.abs(k - k_prev)) > 0.0)

    def w_body(c):
        _, k, it = c
        supp = jnp.dot(k, sup_ref[:, :], preferred_element_type=jnp.float32)
        k_new = jnp.where(supp < 0.5, 1.0, 0.0)
        return k, k_new, it + 1

    _, keep8, _ = jax.lax.while_loop(
        w_cond, w_body, (-ones, ones, jnp.int32(0)))
    keep = keep8[0:1, :]

    valid = jnp.where(scorec != -jnp.inf, 1.0, 0.0)
    keep_f = keep * valid

    ri = jax.lax.broadcasted_iota(jnp.int32, (_NMS_PRE, _NMS_PRE), 0)
    ci = jax.lax.broadcasted_iota(jnp.int32, (_NMS_PRE, _NMS_PRE), 1)
    tri = jnp.where(ri <= ci, 1.0, 0.0)
    pos = jnp.dot(keep_f, tri, preferred_element_type=jnp.float32)

    out_r = jax.lax.broadcasted_iota(jnp.int32, (_NMS_POST, _NMS_PRE), 0)
    posi = (pos - 1.0).astype(jnp.int32)
    sel = jnp.where(posi == out_r, 1.0, 0.0) * keep_f

    score_clean = jnp.where(a[:, 7:8] != -jnp.inf, a[:, 7:8], 0.0)
    a_mm = jnp.concatenate([a[:, 0:7], score_clean], axis=1)
    o_ref[0] = jnp.dot(sel, a_mm, preferred_element_type=jnp.float32)


@functools.partial(jax.jit, static_argnames=("interpret",))
def _nms_pallas(a, at, interpret=False):
    return pl.pallas_call(
        _nms_body,
        grid=(_B,),
        in_specs=[
            pl.BlockSpec((1, _NMS_PRE, 8), lambda b: (b, 0, 0)),
            pl.BlockSpec((1, 8, _NMS_PRE), lambda b: (b, 0, 0)),
        ],
        out_specs=pl.BlockSpec((1, _NMS_POST, 8), lambda b: (b, 0, 0)),
        out_shape=jax.ShapeDtypeStruct((_B, _NMS_POST, 8), jnp.float32),
        scratch_shapes=[
            pltpu.VMEM((_NMS_PRE, _NMS_PRE), jnp.float32),
        ],
        interpret=interpret,
    )(a, at)


def kernel(batch_box_preds, batch_cls_scores, batch_index, batch_size, interpret=False):
    n = batch_cls_scores.shape[0]
    pad_b = jnp.full((_NMS_PRE,), 127, batch_index.dtype)
    pad_s = jnp.full((_NMS_PRE,), jnp.inf, jnp.float32)
    pad_i = jnp.zeros((_NMS_PRE,), jnp.int32)
    bi_p = jnp.concatenate([batch_index, pad_b])
    ns_p = jnp.concatenate([-batch_cls_scores, pad_s])
    ix_p = jnp.concatenate([jnp.arange(n, dtype=jnp.int32), pad_i])
    _, s_neg, s_idx = jax.lax.sort((bi_p, ns_p, ix_p), num_keys=2, is_stable=True)

    bids = jnp.arange(_B + 1, dtype=batch_index.dtype)
    starts = jnp.searchsorted(batch_index, bids).astype(jnp.int32)
    lane = jnp.arange(_NMS_PRE, dtype=jnp.int32)

    tops, idxs = [], []
    for b in range(_B):
        sc = -jax.lax.dynamic_slice(s_neg, (starts[b],), (_NMS_PRE,))
        ix = jax.lax.dynamic_slice(s_idx, (starts[b],), (_NMS_PRE,))
        m = (lane < (starts[b + 1] - starts[b])) & (b < batch_size)
        tops.append(jnp.where(m, sc, -jnp.inf))
        idxs.append(ix)
    top_scores = jnp.stack(tops)
    top_idx = jnp.stack(idxs)
    boxes_sel = jnp.take(batch_box_preds, top_idx.reshape(-1), axis=0)
    boxes_sel = boxes_sel.reshape(_B, _NMS_PRE, 7)
    a = jnp.concatenate([boxes_sel, top_scores[..., None]], axis=-1)
    at = jnp.swapaxes(a, 1, 2)
    return _nms_pallas(a, at, interpret=interpret)
